# baseline (speedup 1.0000x reference)
import functools

import jax
import jax.numpy as jnp
from jax.experimental import pallas as pl
from jax.experimental.pallas import tpu as pltpu

_R = 64
_K = 8
_SCALING = 16.0 / 64.0
_BLK = 512


def _body(x_ref, at_ref, bt_ref, rwt_ref, o_ref):
    x = x_ref[...].astype(jnp.bfloat16)
    h = jnp.dot(x, at_ref[...], preferred_element_type=jnp.float32)
    lg = jnp.dot(h.astype(jnp.bfloat16), rwt_ref[...],
                 preferred_element_type=jnp.float32)
    li = lg[:, :, None]
    lj = lg[:, None, :]
    j_idx = jax.lax.broadcasted_iota(jnp.int32, (1, _R, _R), 2)
    i_idx = jax.lax.broadcasted_iota(jnp.int32, (1, _R, _R), 1)
    beats = (lj > li) | ((lj == li) & (j_idx < i_idx))
    rank = jnp.sum(beats.astype(jnp.int32), axis=-1)
    keep = rank < _K
    hs = jnp.where(keep, h, 0.0).astype(jnp.bfloat16)
    o_ref[...] = jnp.dot(hs, bt_ref[...],
                         preferred_element_type=jnp.float32) * _SCALING


@functools.partial(jax.jit, static_argnames=())
def kernel(x, A, B, Rw):
    bsz, seq, d = x.shape
    n = bsz * seq
    x2 = x.reshape(n, d)
    at = A.T.astype(jnp.bfloat16)
    bt = B.T.astype(jnp.bfloat16)
    rwt = Rw.T.astype(jnp.bfloat16)
    out = pl.pallas_call(
        _body,
        grid=(n // _BLK,),
        in_specs=[
            pl.BlockSpec((_BLK, d), lambda i: (i, 0)),
            pl.BlockSpec((d, _R), lambda i: (0, 0)),
            pl.BlockSpec((_R, d), lambda i: (0, 0)),
            pl.BlockSpec((_R, _R), lambda i: (0, 0)),
        ],
        out_specs=pl.BlockSpec((_BLK, d), lambda i: (i, 0)),
        out_shape=jax.ShapeDtypeStruct((n, d), jnp.float32),
        compiler_params=pltpu.CompilerParams(
            dimension_semantics=("arbitrary",)),
    )(x2, at, bt, rwt)
    return out.reshape(bsz, seq, d)


# speedup vs baseline: 3.6111x; 3.6111x over previous
import functools

import jax
import jax.numpy as jnp
from jax.experimental import pallas as pl
from jax.experimental.pallas import tpu as pltpu

_R = 64
_K = 8
_SCALING = 16.0 / 64.0
_BLK = 512


def _body(x_ref, at_ref, bt_ref, rwt_ref, o_ref):
    x = x_ref[...].astype(jnp.bfloat16)
    h = jnp.dot(x, at_ref[...], preferred_element_type=jnp.float32)
    h16 = h.astype(jnp.bfloat16)
    lg = jnp.dot(h16, rwt_ref[...], preferred_element_type=jnp.float32)
    lanes = jax.lax.broadcasted_iota(jnp.int32, lg.shape, 1)
    cur = lg
    keep = jnp.zeros(lg.shape, jnp.bool_)
    for _ in range(_K):
        m = jnp.max(cur, axis=1, keepdims=True)
        cand = cur == m
        sel_idx = jnp.min(jnp.where(cand, lanes, _R), axis=1, keepdims=True)
        sel = lanes == sel_idx
        keep = keep | sel
        cur = jnp.where(sel, -jnp.inf, cur)
    hs = jnp.where(keep, h16, jnp.bfloat16(0.0))
    o_ref[...] = jnp.dot(hs, bt_ref[...],
                         preferred_element_type=jnp.float32) * _SCALING


@functools.partial(jax.jit, static_argnames=())
def kernel(x, A, B, Rw):
    bsz, seq, d = x.shape
    n = bsz * seq
    x2 = x.reshape(n, d)
    at = A.T.astype(jnp.bfloat16)
    bt = B.T.astype(jnp.bfloat16)
    rwt = Rw.T.astype(jnp.bfloat16)
    out = pl.pallas_call(
        _body,
        grid=(n // _BLK,),
        in_specs=[
            pl.BlockSpec((_BLK, d), lambda i: (i, 0)),
            pl.BlockSpec((d, _R), lambda i: (0, 0)),
            pl.BlockSpec((_R, d), lambda i: (0, 0)),
            pl.BlockSpec((_R, _R), lambda i: (0, 0)),
        ],
        out_specs=pl.BlockSpec((_BLK, d), lambda i: (i, 0)),
        out_shape=jax.ShapeDtypeStruct((n, d), jnp.float32),
        compiler_params=pltpu.CompilerParams(
            dimension_semantics=("parallel",)),
    )(x2, at, bt, rwt)
    return out.reshape(bsz, seq, d)


# speedup vs baseline: 3.6115x; 1.0001x over previous
import functools

import jax
import jax.numpy as jnp
from jax.experimental import pallas as pl
from jax.experimental.pallas import tpu as pltpu

_R = 64
_K = 8
_SCALING = 16.0 / 64.0
_BLK = 512


def _body(x_ref, at_ref, bt_ref, rwt_ref, o_ref):
    x = x_ref[...].astype(jnp.bfloat16)
    h = jnp.dot(x, at_ref[...], preferred_element_type=jnp.float32)
    h16 = h.astype(jnp.bfloat16)
    lg = jnp.dot(h16, rwt_ref[...], preferred_element_type=jnp.float32)
    lanes = jax.lax.broadcasted_iota(jnp.int32, lg.shape, 1)
    cur = lg
    keep = jnp.zeros(lg.shape, jnp.bool_)
    for _ in range(_K):
        m = jnp.max(cur, axis=1, keepdims=True)
        cand = cur == m
        sel_idx = jnp.min(jnp.where(cand, lanes, _R), axis=1, keepdims=True)
        sel = lanes == sel_idx
        keep = keep | sel
        cur = jnp.where(sel, -jnp.inf, cur)
    hs = jnp.where(keep, h16, jnp.bfloat16(0.0))
    o_ref[...] = jnp.dot(hs, bt_ref[...],
                         preferred_element_type=jnp.float32)


@functools.partial(jax.jit, static_argnames=())
def kernel(x, A, B, Rw):
    bsz, seq, d = x.shape
    n = bsz * seq
    x2 = x.reshape(n, d)
    at = A.T.astype(jnp.bfloat16)
    bt = (B.T * _SCALING).astype(jnp.bfloat16)
    rwt = Rw.T.astype(jnp.bfloat16)
    out = pl.pallas_call(
        _body,
        grid=(n // _BLK,),
        in_specs=[
            pl.BlockSpec((_BLK, d), lambda i: (i, 0)),
            pl.BlockSpec((d, _R), lambda i: (0, 0)),
            pl.BlockSpec((_R, d), lambda i: (0, 0)),
            pl.BlockSpec((_R, _R), lambda i: (0, 0)),
        ],
        out_specs=pl.BlockSpec((_BLK, d), lambda i: (i, 0)),
        out_shape=jax.ShapeDtypeStruct((n, d), jnp.float32),
        compiler_params=pltpu.CompilerParams(
            dimension_semantics=("parallel",)),
    )(x2, at, bt, rwt)
    return out.reshape(bsz, seq, d)
